# baseline (speedup 1.0000x reference)
import functools

import jax
import jax.numpy as jnp
from jax import lax
from jax.experimental import pallas as pl
from jax.experimental.pallas import tpu as pltpu
from jax.experimental.pallas import tpu_sc as plsc

B, T, H = 1024, 200, 64
V = 100000
BT = B * T
NC, NS = 2, 16
NW = NC * NS
RPW = BT // NW
CH = 128
NCHUNK = RPW // CH
SB = B // NW

_MESH = plsc.VectorSubcoreMesh(core_axis_name="c", subcore_axis_name="s",
                               num_cores=NC, num_subcores=NS)


@functools.partial(
    pl.kernel,
    out_type=(
        jax.ShapeDtypeStruct((BT, 7 * H), jnp.float32),
        jax.ShapeDtypeStruct((BT, 8 * H), jnp.float32),
        jax.ShapeDtypeStruct((B, 7 * H), jnp.float32),
    ),
    mesh=_MESH,
    compiler_params=pltpu.CompilerParams(use_tc_tiling_on_sc=False),
    scratch_types=[
        pltpu.VMEM((NCHUNK, CH), jnp.int32),
        pltpu.VMEM((SB,), jnp.int32),
        pltpu.VMEM((CH, H), jnp.float32),
        pltpu.VMEM((SB, H), jnp.float32),
        pltpu.SemaphoreType.DMA,
    ],
)
def _sc_gather(kidx, oidx, sidx, ktab, otab, stab,
               tk_out, tob_out, sin_out,
               idx_v, sidx_v, rv0, srv, sem0):
    wid = lax.axis_index("s") * NC + lax.axis_index("c")
    row0 = wid * RPW

    for i in range(3):
        pltpu.sync_copy(sidx.at[i, wid], sidx_v)
        pltpu.async_copy(stab.at[sidx_v], srv, sem0).wait()
        pltpu.sync_copy(srv, sin_out.at[pl.ds(wid * SB, SB),
                                        pl.ds(i * H, H)])

    def run_field(tab, idx_slice, out, col0):
        pltpu.sync_copy(idx_slice, idx_v)

        def body(g, c):
            pltpu.async_copy(tab.at[idx_v.at[g]], rv0, sem0).wait()
            pltpu.sync_copy(rv0, out.at[pl.ds(row0 + g * CH, CH),
                                        pl.ds(col0, H)])
            return c

        lax.fori_loop(0, NCHUNK, body, 0)

    for i in range(3):
        run_field(ktab, kidx.at[i, wid], tk_out, i * H)
    for i in range(2):
        run_field(otab, oidx.at[i, wid], tob_out, i * H)


def _cont_body(prev_ref, vals_ref, emb_ref, bias_ref, out_ref):
    out_ref[...] = (vals_ref[0, 0][:, None] * emb_ref[0, 0][None, :]
                    + bias_ref[0, 0][None, :])[:, None, None, :]


def _cont_fill(buf, vals_t, emb, bias, ncat):
    n = buf.shape[0]
    nf = emb.shape[0]
    f_total = buf.shape[1] // H
    R = 2048 if n % 2048 == 0 else n
    out = pl.pallas_call(
        _cont_body,
        grid=(n // R, nf),
        in_specs=[
            pl.BlockSpec(memory_space=pl.ANY),
            pl.BlockSpec((1, 1, R), lambda i, j: (j, 0, i)),
            pl.BlockSpec((1, 1, H), lambda i, j: (j, 0, 0)),
            pl.BlockSpec((1, 1, H), lambda i, j: (j, 0, 0)),
        ],
        out_specs=pl.BlockSpec((R, 1, 1, H),
                               lambda i, j, _n=ncat: (i, _n + j, 0, 0)),
        out_shape=jax.ShapeDtypeStruct((n, f_total, 1, H), jnp.float32),
        input_output_aliases={0: 0},
    )(buf.reshape(n, f_total, 1, H), vals_t.reshape(nf, 1, n),
      emb.reshape(nf, 1, H), bias.reshape(nf, 1, H))
    return out.reshape(n, f_total * H)


def _tgt_body(vals_ref, emb_ref, bias_ref, out_ref):
    out_ref[...] = (vals_ref[0][:, None] * emb_ref[0][None, :]
                    + bias_ref[0][None, :])


def _tgt_fill(vals_t, emb, bias):
    R = 2048
    return pl.pallas_call(
        _tgt_body,
        grid=(BT // R,),
        in_specs=[
            pl.BlockSpec((1, R), lambda i: (0, i)),
            pl.BlockSpec((1, H), lambda i: (0, 0)),
            pl.BlockSpec((1, H), lambda i: (0, 0)),
        ],
        out_specs=pl.BlockSpec((R, H), lambda i: (i, 0)),
        out_shape=jax.ShapeDtypeStruct((BT, H), jnp.float32),
    )(vals_t, emb, bias)


def kernel(s_cat, s_cont, k_cat, k_cont, o_cat, o_cont, target,
           s_cat_tables, k_cat_tables, o_cat_tables,
           s_cont_emb, s_cont_bias, k_cont_emb, k_cont_bias,
           o_cont_emb, o_cont_bias, tgt_emb, tgt_bias):
    koff = jnp.arange(3, dtype=jnp.int32) * V
    ooff = jnp.arange(2, dtype=jnp.int32) * V
    kidx = (k_cat.reshape(BT, 3) + koff).T.reshape(3, NW, NCHUNK, CH)
    oidx = (o_cat.reshape(BT, 2) + ooff).T.reshape(2, NW, NCHUNK, CH)
    sidx = (s_cat[:, 0, :] + koff).T.reshape(3, NW, SB)
    ktab = k_cat_tables.reshape(3 * V, H)
    otab = o_cat_tables.reshape(2 * V, H)
    stab = s_cat_tables.reshape(3 * V, H)

    tk0, tob0, sin0 = _sc_gather(kidx, oidx, sidx, ktab, otab, stab)

    tk = _cont_fill(tk0, k_cont.reshape(BT, 4).T, k_cont_emb, k_cont_bias, 3)
    tob = _cont_fill(tob0, o_cont.reshape(BT, 6).T, o_cont_emb, o_cont_bias, 2)
    sin = _cont_fill(sin0, s_cont[:, 0, :].T, s_cont_emb, s_cont_bias, 3)
    ttgt = _tgt_fill(target.reshape(1, BT), tgt_emb, tgt_bias)

    return (sin.reshape(B, 7, H), tk.reshape(B, T, 7, H),
            tob.reshape(B, T, 8, H), ttgt.reshape(B, T, 1, H))


# speedup vs baseline: 5.2932x; 5.2932x over previous
import functools

import jax
import jax.numpy as jnp
from jax import lax
from jax.experimental import pallas as pl
from jax.experimental.pallas import tpu as pltpu
from jax.experimental.pallas import tpu_sc as plsc

B, T, H = 1024, 200, 64
V = 100000
BT = B * T
NC, NS = 2, 16
NW = NC * NS
RPW = BT // NW
CH = 128
NCHUNK = RPW // CH
SB = B // NW
HB = 2 * H

_MESH = plsc.VectorSubcoreMesh(core_axis_name="c", subcore_axis_name="s",
                               num_cores=NC, num_subcores=NS)


@functools.partial(
    pl.kernel,
    out_type=(
        jax.ShapeDtypeStruct((3, BT // 2, HB), jnp.float32),
        jax.ShapeDtypeStruct((2, BT // 2, HB), jnp.float32),
        jax.ShapeDtypeStruct((3, B // 2, HB), jnp.float32),
    ),
    mesh=_MESH,
    compiler_params=pltpu.CompilerParams(use_tc_tiling_on_sc=False),
    scratch_types=[
        pltpu.VMEM((NCHUNK, CH), jnp.int32),
        pltpu.VMEM((SB,), jnp.int32),
        pltpu.VMEM((CH, H), jnp.float32),
        pltpu.VMEM((SB, H), jnp.float32),
        pltpu.SemaphoreType.DMA,
    ],
)
def _sc_gather(kidx, oidx, sidx, ktab, otab, stab,
               kst_out, ost_out, sst_out,
               idx_v, sidx_v, rv0, srv, sem0):
    wid = lax.axis_index("s") * NC + lax.axis_index("c")
    row0 = wid * RPW

    def pack_dst(out, i, p0, n):
        t_ = p0 // B
        rem = p0 - t_ * B
        half = rem // (B // 2)
        u = rem - half * (B // 2)
        return out.at[i, pl.ds(t_ * (B // 2) + u, n), pl.ds(half * H, H)]

    for i in range(3):
        pltpu.sync_copy(sidx.at[i, wid], sidx_v)
        pltpu.async_copy(stab.at[sidx_v], srv, sem0).wait()
        pltpu.sync_copy(srv, pack_dst(sst_out, i, wid * SB, SB))

    def run_field(tab, idx_slice, out, i):
        pltpu.sync_copy(idx_slice, idx_v)

        def body(g, c):
            pltpu.async_copy(tab.at[idx_v.at[g]], rv0, sem0).wait()
            pltpu.sync_copy(rv0, pack_dst(out, i, row0 + g * CH, CH))
            return c

        lax.fori_loop(0, NCHUNK, body, 0)

    for i in range(3):
        run_field(ktab, kidx.at[i, wid], kst_out, i)
    for i in range(2):
        run_field(otab, oidx.at[i, wid], ost_out, i)


def _unpack_t(s):
    return jnp.concatenate([s[:, :H].T, s[:, H:].T], axis=1)


def _assemble_body(ncat, stage_ref, vals_ref, emb_ref, bias_ref, out_ref):
    for i in range(ncat):
        out_ref[0, i] = _unpack_t(stage_ref[i, 0])
    nf = emb_ref.shape[0]
    for j in range(nf):
        out_ref[0, ncat + j] = (emb_ref[j][:, None]
                                * vals_ref[j, 0, 0][None, :]
                                + bias_ref[j][:, None])


def _assemble(stage, vals_t, emb, bias, ncat):
    nf = emb.shape[0]
    f_total = ncat + nf
    return pl.pallas_call(
        functools.partial(_assemble_body, ncat),
        grid=(T,),
        in_specs=[
            pl.BlockSpec((ncat, 1, B // 2, HB), lambda i: (0, i, 0, 0)),
            pl.BlockSpec((nf, 1, 1, B), lambda i: (0, i, 0, 0)),
            pl.BlockSpec((nf, H), lambda i: (0, 0)),
            pl.BlockSpec((nf, H), lambda i: (0, 0)),
        ],
        out_specs=pl.BlockSpec((1, f_total, H, B), lambda i: (i, 0, 0, 0)),
        out_shape=jax.ShapeDtypeStruct((T, f_total, H, B), jnp.float32),
    )(stage, vals_t.reshape(nf, T, 1, B), emb, bias)


def _sinp_body(stage_ref, vals_ref, emb_ref, bias_ref, out_ref):
    for i in range(3):
        out_ref[i] = _unpack_t(stage_ref[i])
    for j in range(4):
        out_ref[3 + j] = (emb_ref[j][:, None] * vals_ref[j][None, :]
                          + bias_ref[j][:, None])


def _sinp_assemble(stage, vals_t, emb, bias):
    return pl.pallas_call(
        _sinp_body,
        grid=(1,),
        in_specs=[
            pl.BlockSpec((3, B // 2, HB), lambda i: (0, 0, 0)),
            pl.BlockSpec((4, B), lambda i: (0, 0)),
            pl.BlockSpec((4, H), lambda i: (0, 0)),
            pl.BlockSpec((4, H), lambda i: (0, 0)),
        ],
        out_specs=pl.BlockSpec((7, H, B), lambda i: (0, 0, 0)),
        out_shape=jax.ShapeDtypeStruct((7, H, B), jnp.float32),
    )(stage, vals_t, emb, bias)


def _tgt_body(vals_ref, emb_ref, bias_ref, out_ref):
    out_ref[0, 0] = (emb_ref[0][:, None] * vals_ref[0, 0, 0][None, :]
                     + bias_ref[0][:, None])


def _tgt_fill(vals_t, emb, bias):
    return pl.pallas_call(
        _tgt_body,
        grid=(T,),
        in_specs=[
            pl.BlockSpec((1, 1, 1, B), lambda i: (0, i, 0, 0)),
            pl.BlockSpec((1, H), lambda i: (0, 0)),
            pl.BlockSpec((1, H), lambda i: (0, 0)),
        ],
        out_specs=pl.BlockSpec((1, 1, H, B), lambda i: (i, 0, 0, 0)),
        out_shape=jax.ShapeDtypeStruct((T, 1, H, B), jnp.float32),
    )(vals_t.reshape(1, T, 1, B), emb, bias)


def kernel(s_cat, s_cont, k_cat, k_cont, o_cat, o_cont, target,
           s_cat_tables, k_cat_tables, o_cat_tables,
           s_cont_emb, s_cont_bias, k_cont_emb, k_cont_bias,
           o_cont_emb, o_cont_bias, tgt_emb, tgt_bias):
    koff = jnp.arange(3, dtype=jnp.int32) * V
    ooff = jnp.arange(2, dtype=jnp.int32) * V
    kidx = (k_cat + koff).transpose(2, 1, 0).reshape(3, NW, NCHUNK, CH)
    oidx = (o_cat + ooff).transpose(2, 1, 0).reshape(2, NW, NCHUNK, CH)
    sidx = (s_cat[:, 0, :] + koff).T.reshape(3, NW, SB)
    ktab = k_cat_tables.reshape(3 * V, H)
    otab = o_cat_tables.reshape(2 * V, H)
    stab = s_cat_tables.reshape(3 * V, H)

    kst, ost, sst = _sc_gather(kidx, oidx, sidx, ktab, otab, stab)

    kvals = k_cont.transpose(2, 1, 0)
    ovals = o_cont.transpose(2, 1, 0)
    svals = s_cont[:, 0, :].T
    tvals = target.transpose(2, 1, 0)

    tk = _assemble(kst.reshape(3, T, B // 2, HB), kvals,
                   k_cont_emb, k_cont_bias, 3)
    tob = _assemble(ost.reshape(2, T, B // 2, HB), ovals,
                    o_cont_emb, o_cont_bias, 2)
    sin = _sinp_assemble(sst, svals, s_cont_emb, s_cont_bias)
    ttgt = _tgt_fill(tvals, tgt_emb, tgt_bias)

    return (sin.transpose(2, 0, 1),
            tk.transpose(3, 0, 1, 2),
            tob.transpose(3, 0, 1, 2),
            ttgt.transpose(3, 0, 1, 2))


# speedup vs baseline: 6.0432x; 1.1417x over previous
import functools

import jax
import jax.numpy as jnp
from jax import lax
from jax.experimental import pallas as pl
from jax.experimental.pallas import tpu as pltpu
from jax.experimental.pallas import tpu_sc as plsc

B, T, H = 1024, 200, 64
V = 100000
BT = B * T
NC, NS = 2, 16
NW = NC * NS
RPW = BT // NW
CH = 128
NCHUNK = RPW // CH
SB = B // NW
HB = 2 * H

_MESH = plsc.VectorSubcoreMesh(core_axis_name="c", subcore_axis_name="s",
                               num_cores=NC, num_subcores=NS)


def _pack_dst(out, i, p0, n):
    t_ = p0 // B
    rem = p0 - t_ * B
    half = rem // (B // 2)
    u = rem - half * (B // 2)
    return out.at[i, pl.ds(t_ * (B // 2) + u, n), pl.ds(half * H, H)]


def _run_field(tab, idx_slice, out, i, row0, idx_v, rv0, rv1, sem0, sem1):
    pltpu.sync_copy(idx_slice, idx_v)

    def start(g, rv, sem):
        pltpu.async_copy(tab.at[idx_v.at[g]], rv, sem)

    def drain(rv, sem):
        pltpu.make_async_copy(tab.at[pl.ds(0, CH)], rv, sem).wait()

    def store(g, rv):
        pltpu.sync_copy(rv, _pack_dst(out, i, row0 + g * CH, CH))

    start(0, rv0, sem0)

    def pair(it, c):
        g0 = 2 * it
        start(g0 + 1, rv1, sem1)
        drain(rv0, sem0)
        store(g0, rv0)
        start(g0 + 2, rv0, sem0)
        drain(rv1, sem1)
        store(g0 + 1, rv1)
        return c

    lax.fori_loop(0, NCHUNK // 2 - 1, pair, 0)
    g = NCHUNK - 2
    start(g + 1, rv1, sem1)
    drain(rv0, sem0)
    store(g, rv0)
    drain(rv1, sem1)
    store(g + 1, rv1)


_SC_SCRATCH = [
    pltpu.VMEM((NCHUNK, CH), jnp.int32),
    pltpu.VMEM((CH, H), jnp.float32),
    pltpu.VMEM((CH, H), jnp.float32),
    pltpu.SemaphoreType.DMA,
    pltpu.SemaphoreType.DMA,
]


@functools.partial(
    pl.kernel,
    out_type=jax.ShapeDtypeStruct((3, BT // 2, HB), jnp.float32),
    mesh=_MESH,
    compiler_params=pltpu.CompilerParams(use_tc_tiling_on_sc=False),
    scratch_types=_SC_SCRATCH,
)
def _sc_gather_k(kidx, ktab, kst_out, idx_v, rv0, rv1, sem0, sem1):
    wid = lax.axis_index("s") * NC + lax.axis_index("c")
    row0 = wid * RPW
    for i in range(3):
        _run_field(ktab, kidx.at[i, wid], kst_out, i, row0,
                   idx_v, rv0, rv1, sem0, sem1)


@functools.partial(
    pl.kernel,
    out_type=(
        jax.ShapeDtypeStruct((2, BT // 2, HB), jnp.float32),
        jax.ShapeDtypeStruct((3, B // 2, HB), jnp.float32),
    ),
    mesh=_MESH,
    compiler_params=pltpu.CompilerParams(use_tc_tiling_on_sc=False),
    scratch_types=_SC_SCRATCH + [
        pltpu.VMEM((SB,), jnp.int32),
        pltpu.VMEM((SB, H), jnp.float32),
    ],
)
def _sc_gather_os(oidx, sidx, otab, stab, ost_out, sst_out,
                  idx_v, rv0, rv1, sem0, sem1, sidx_v, srv):
    wid = lax.axis_index("s") * NC + lax.axis_index("c")
    row0 = wid * RPW

    for i in range(3):
        pltpu.sync_copy(sidx.at[i, wid], sidx_v)
        pltpu.async_copy(stab.at[sidx_v], srv, sem0).wait()
        pltpu.sync_copy(srv, _pack_dst(sst_out, i, wid * SB, SB))

    for i in range(2):
        _run_field(otab, oidx.at[i, wid], ost_out, i, row0,
                   idx_v, rv0, rv1, sem0, sem1)


def _unpack_t(s):
    return jnp.concatenate([s[:, :H].T, s[:, H:].T], axis=1)


def _assemble_body(ncat, stage_ref, vals_ref, emb_ref, bias_ref, out_ref):
    for i in range(ncat):
        out_ref[0, i] = _unpack_t(stage_ref[i, 0])
    nf = emb_ref.shape[0]
    for j in range(nf):
        out_ref[0, ncat + j] = (emb_ref[j][:, None]
                                * vals_ref[j, 0, 0][None, :]
                                + bias_ref[j][:, None])


def _assemble(stage, vals_t, emb, bias, ncat):
    nf = emb.shape[0]
    f_total = ncat + nf
    return pl.pallas_call(
        functools.partial(_assemble_body, ncat),
        grid=(T,),
        in_specs=[
            pl.BlockSpec((ncat, 1, B // 2, HB), lambda i: (0, i, 0, 0)),
            pl.BlockSpec((nf, 1, 1, B), lambda i: (0, i, 0, 0)),
            pl.BlockSpec((nf, H), lambda i: (0, 0)),
            pl.BlockSpec((nf, H), lambda i: (0, 0)),
        ],
        out_specs=pl.BlockSpec((1, f_total, H, B), lambda i: (i, 0, 0, 0)),
        out_shape=jax.ShapeDtypeStruct((T, f_total, H, B), jnp.float32),
    )(stage, vals_t.reshape(nf, T, 1, B), emb, bias)


def _sinp_body(stage_ref, vals_ref, emb_ref, bias_ref, out_ref):
    for i in range(3):
        out_ref[i] = _unpack_t(stage_ref[i])
    for j in range(4):
        out_ref[3 + j] = (emb_ref[j][:, None] * vals_ref[j][None, :]
                          + bias_ref[j][:, None])


def _sinp_assemble(stage, vals_t, emb, bias):
    return pl.pallas_call(
        _sinp_body,
        grid=(1,),
        in_specs=[
            pl.BlockSpec((3, B // 2, HB), lambda i: (0, 0, 0)),
            pl.BlockSpec((4, B), lambda i: (0, 0)),
            pl.BlockSpec((4, H), lambda i: (0, 0)),
            pl.BlockSpec((4, H), lambda i: (0, 0)),
        ],
        out_specs=pl.BlockSpec((7, H, B), lambda i: (0, 0, 0)),
        out_shape=jax.ShapeDtypeStruct((7, H, B), jnp.float32),
    )(stage, vals_t, emb, bias)


def _tgt_body(vals_ref, emb_ref, bias_ref, out_ref):
    out_ref[0, 0] = (emb_ref[0][:, None] * vals_ref[0, 0, 0][None, :]
                     + bias_ref[0][:, None])


def _tgt_fill(vals_t, emb, bias):
    return pl.pallas_call(
        _tgt_body,
        grid=(T,),
        in_specs=[
            pl.BlockSpec((1, 1, 1, B), lambda i: (0, i, 0, 0)),
            pl.BlockSpec((1, H), lambda i: (0, 0)),
            pl.BlockSpec((1, H), lambda i: (0, 0)),
        ],
        out_specs=pl.BlockSpec((1, 1, H, B), lambda i: (i, 0, 0, 0)),
        out_shape=jax.ShapeDtypeStruct((T, 1, H, B), jnp.float32),
    )(vals_t.reshape(1, T, 1, B), emb, bias)


def kernel(s_cat, s_cont, k_cat, k_cont, o_cat, o_cont, target,
           s_cat_tables, k_cat_tables, o_cat_tables,
           s_cont_emb, s_cont_bias, k_cont_emb, k_cont_bias,
           o_cont_emb, o_cont_bias, tgt_emb, tgt_bias):
    koff = jnp.arange(3, dtype=jnp.int32) * V
    ooff = jnp.arange(2, dtype=jnp.int32) * V
    kidx = (k_cat + koff).transpose(2, 1, 0).reshape(3, NW, NCHUNK, CH)
    oidx = (o_cat + ooff).transpose(2, 1, 0).reshape(2, NW, NCHUNK, CH)
    sidx = (s_cat[:, 0, :] + koff).T.reshape(3, NW, SB)
    ktab = k_cat_tables.reshape(3 * V, H)
    otab = o_cat_tables.reshape(2 * V, H)
    stab = s_cat_tables.reshape(3 * V, H)

    kst = _sc_gather_k(kidx, ktab)
    ost, sst = _sc_gather_os(oidx, sidx, otab, stab)

    kvals = k_cont.transpose(2, 1, 0)
    ovals = o_cont.transpose(2, 1, 0)
    svals = s_cont[:, 0, :].T
    tvals = target.transpose(2, 1, 0)

    tk = _assemble(kst.reshape(3, T, B // 2, HB), kvals,
                   k_cont_emb, k_cont_bias, 3)
    tob = _assemble(ost.reshape(2, T, B // 2, HB), ovals,
                    o_cont_emb, o_cont_bias, 2)
    sin = _sinp_assemble(sst, svals, s_cont_emb, s_cont_bias)
    ttgt = _tgt_fill(tvals, tgt_emb, tgt_bias)

    return (sin.transpose(2, 0, 1),
            tk.transpose(3, 0, 1, 2),
            tob.transpose(3, 0, 1, 2),
            ttgt.transpose(3, 0, 1, 2))


# speedup vs baseline: 6.0631x; 1.0033x over previous
import functools

import jax
import jax.numpy as jnp
from jax import lax
from jax.experimental import pallas as pl
from jax.experimental.pallas import tpu as pltpu
from jax.experimental.pallas import tpu_sc as plsc

B, T, H = 1024, 200, 64
V = 100000
BT = B * T
NC, NS = 2, 16
NW = NC * NS
RPW = BT // NW
CH = 128
NCHUNK = RPW // CH
SB = B // NW
HB = 2 * H

_MESH = plsc.VectorSubcoreMesh(core_axis_name="c", subcore_axis_name="s",
                               num_cores=NC, num_subcores=NS)


def _pack_dst(out, i, p0, n):
    t_ = p0 // B
    rem = p0 - t_ * B
    half = rem // (B // 2)
    u = rem - half * (B // 2)
    return out.at[i, pl.ds(t_ * (B // 2) + u, n), pl.ds(half * H, H)]


def _run_field(tab, idx_slice, out, i, row0, idx_v, rv0, rv1, sem0, sem1):
    pltpu.sync_copy(idx_slice, idx_v)

    def start(g, rv, sem):
        pltpu.async_copy(tab.at[idx_v.at[g]], rv, sem)

    def drain(rv, sem):
        pltpu.make_async_copy(tab.at[pl.ds(0, CH)], rv, sem).wait()

    def store(g, rv):
        pltpu.sync_copy(rv, _pack_dst(out, i, row0 + g * CH, CH))

    start(0, rv0, sem0)

    def pair(it, c):
        g0 = 2 * it
        start(g0 + 1, rv1, sem1)
        drain(rv0, sem0)
        store(g0, rv0)
        start(g0 + 2, rv0, sem0)
        drain(rv1, sem1)
        store(g0 + 1, rv1)
        return c

    lax.fori_loop(0, NCHUNK // 2 - 1, pair, 0)
    g = NCHUNK - 2
    start(g + 1, rv1, sem1)
    drain(rv0, sem0)
    store(g, rv0)
    drain(rv1, sem1)
    store(g + 1, rv1)


_SC_SCRATCH = [
    pltpu.VMEM((NCHUNK, CH), jnp.int32),
    pltpu.VMEM((CH, H), jnp.float32),
    pltpu.VMEM((CH, H), jnp.float32),
    pltpu.SemaphoreType.DMA,
    pltpu.SemaphoreType.DMA,
]


@functools.partial(
    pl.kernel,
    out_type=jax.ShapeDtypeStruct((3, BT // 2, HB), jnp.float32),
    mesh=_MESH,
    compiler_params=pltpu.CompilerParams(use_tc_tiling_on_sc=False),
    scratch_types=_SC_SCRATCH,
)
def _sc_gather_k(kidx, ktab, kst_out, idx_v, rv0, rv1, sem0, sem1):
    wid = lax.axis_index("s") * NC + lax.axis_index("c")
    row0 = wid * RPW
    for i in range(3):
        _run_field(ktab, kidx.at[i, wid], kst_out, i, row0,
                   idx_v, rv0, rv1, sem0, sem1)


@functools.partial(
    pl.kernel,
    out_type=(
        jax.ShapeDtypeStruct((2, BT // 2, HB), jnp.float32),
        jax.ShapeDtypeStruct((3, B // 2, HB), jnp.float32),
    ),
    mesh=_MESH,
    compiler_params=pltpu.CompilerParams(use_tc_tiling_on_sc=False),
    scratch_types=_SC_SCRATCH + [
        pltpu.VMEM((SB,), jnp.int32),
        pltpu.VMEM((SB, H), jnp.float32),
    ],
)
def _sc_gather_os(oidx, sidx, otab, stab, ost_out, sst_out,
                  idx_v, rv0, rv1, sem0, sem1, sidx_v, srv):
    wid = lax.axis_index("s") * NC + lax.axis_index("c")
    row0 = wid * RPW

    for i in range(3):
        pltpu.sync_copy(sidx.at[i, wid], sidx_v)
        pltpu.async_copy(stab.at[sidx_v], srv, sem0).wait()
        pltpu.sync_copy(srv, _pack_dst(sst_out, i, wid * SB, SB))

    for i in range(2):
        _run_field(otab, oidx.at[i, wid], ost_out, i, row0,
                   idx_v, rv0, rv1, sem0, sem1)


def _unpack_t(s):
    return jnp.concatenate([s[:, :H].T, s[:, H:].T], axis=1)


def _assemble_body(ncat, stage_ref, vals_ref, emb_ref, bias_ref, out_ref):
    for i in range(ncat):
        out_ref[0, i] = _unpack_t(stage_ref[i, 0])
    nf = emb_ref.shape[0]
    for j in range(nf):
        out_ref[0, ncat + j] = (emb_ref[j][:, None]
                                * vals_ref[j, 0, 0][None, :]
                                + bias_ref[j][:, None])


def _assemble(stage, vals_t, emb, bias, ncat):
    nf = emb.shape[0]
    f_total = ncat + nf
    return pl.pallas_call(
        functools.partial(_assemble_body, ncat),
        grid=(T,),
        in_specs=[
            pl.BlockSpec((ncat, 1, B // 2, HB), lambda i: (0, i, 0, 0)),
            pl.BlockSpec((nf, 1, 1, B), lambda i: (0, i, 0, 0)),
            pl.BlockSpec((nf, H), lambda i: (0, 0)),
            pl.BlockSpec((nf, H), lambda i: (0, 0)),
        ],
        out_specs=pl.BlockSpec((1, f_total, H, B), lambda i: (i, 0, 0, 0)),
        out_shape=jax.ShapeDtypeStruct((T, f_total, H, B), jnp.float32),
    )(stage, vals_t.reshape(nf, T, 1, B), emb, bias)


def _sinp_body(stage_ref, vals_ref, emb_ref, bias_ref, out_ref):
    for i in range(3):
        out_ref[i] = _unpack_t(stage_ref[i])
    for j in range(4):
        out_ref[3 + j] = (emb_ref[j][:, None] * vals_ref[j][None, :]
                          + bias_ref[j][:, None])


def _sinp_assemble(stage, vals_t, emb, bias):
    return pl.pallas_call(
        _sinp_body,
        grid=(1,),
        in_specs=[
            pl.BlockSpec((3, B // 2, HB), lambda i: (0, 0, 0)),
            pl.BlockSpec((4, B), lambda i: (0, 0)),
            pl.BlockSpec((4, H), lambda i: (0, 0)),
            pl.BlockSpec((4, H), lambda i: (0, 0)),
        ],
        out_specs=pl.BlockSpec((7, H, B), lambda i: (0, 0, 0)),
        out_shape=jax.ShapeDtypeStruct((7, H, B), jnp.float32),
    )(stage, vals_t, emb, bias)


def _tgt_body(vals_ref, emb_ref, bias_ref, out_ref):
    out_ref[0, 0] = (emb_ref[0][:, None] * vals_ref[0, 0, 0][None, :]
                     + bias_ref[0][:, None])


def _tgt_fill(vals_t, emb, bias):
    return pl.pallas_call(
        _tgt_body,
        grid=(T,),
        in_specs=[
            pl.BlockSpec((1, 1, 1, B), lambda i: (0, i, 0, 0)),
            pl.BlockSpec((1, H), lambda i: (0, 0)),
            pl.BlockSpec((1, H), lambda i: (0, 0)),
        ],
        out_specs=pl.BlockSpec((1, 1, H, B), lambda i: (i, 0, 0, 0)),
        out_shape=jax.ShapeDtypeStruct((T, 1, H, B), jnp.float32),
    )(vals_t.reshape(1, T, 1, B), emb, bias)


def kernel(s_cat, s_cont, k_cat, k_cont, o_cat, o_cont, target,
           s_cat_tables, k_cat_tables, o_cat_tables,
           s_cont_emb, s_cont_bias, k_cont_emb, k_cont_bias,
           o_cont_emb, o_cont_bias, tgt_emb, tgt_bias):
    koff = jnp.arange(3, dtype=jnp.int32) * V
    ooff = jnp.arange(2, dtype=jnp.int32) * V
    kidx = (k_cat + koff).transpose(2, 1, 0).reshape(3, NW, NCHUNK, CH)
    oidx = (o_cat + ooff).transpose(2, 1, 0).reshape(2, NW, NCHUNK, CH)
    sidx = (s_cat[:, 0, :] + koff).T.reshape(3, NW, SB)
    ktab = k_cat_tables.reshape(3 * V, H)
    otab = o_cat_tables.reshape(2 * V, H)
    stab = s_cat_tables.reshape(3 * V, H)

    kst = _sc_gather_k(kidx, ktab)
    oidx, sidx = lax.optimization_barrier((oidx, sidx, kst))[:2]
    ost, sst = _sc_gather_os(oidx, sidx, otab, stab)

    kvals = k_cont.transpose(2, 1, 0)
    ovals = o_cont.transpose(2, 1, 0)
    svals = s_cont[:, 0, :].T
    tvals = target.transpose(2, 1, 0)

    tk = _assemble(kst.reshape(3, T, B // 2, HB), kvals,
                   k_cont_emb, k_cont_bias, 3)
    tob = _assemble(ost.reshape(2, T, B // 2, HB), ovals,
                    o_cont_emb, o_cont_bias, 2)
    sin = _sinp_assemble(sst, svals, s_cont_emb, s_cont_bias)
    ttgt = _tgt_fill(tvals, tgt_emb, tgt_bias)

    return (sin.transpose(2, 0, 1),
            tk.transpose(3, 0, 1, 2),
            tob.transpose(3, 0, 1, 2),
            ttgt.transpose(3, 0, 1, 2))
